# baseline (speedup 1.0000x reference)
import functools

import jax
import jax.numpy as jnp
from jax import lax
from jax.experimental import pallas as pl
from jax.experimental.pallas import tpu as pltpu
from jax.experimental.pallas import tpu_sc as plsc

_N = 10000
_E = 160000
_D = 128

_NC = 2
_NS = 16
_NW = _NC * _NS


_G_CH = 80
_NBUF = 5


def _sc_gather(table, idx, m):
    bpw = m // _NW
    nch = bpw // _G_CH
    assert bpw % _G_CH == 0 and nch % _NBUF == 0
    mesh = plsc.VectorSubcoreMesh(core_axis_name="c", subcore_axis_name="s")

    @functools.partial(
        pl.kernel,
        out_type=jax.ShapeDtypeStruct((m, _D), jnp.float32),
        mesh=mesh,
        scratch_types=(
            [pltpu.VMEM((bpw,), jnp.int32)]
            + [pltpu.VMEM((_G_CH, _D), jnp.float32)] * _NBUF
            + [pltpu.SemaphoreType.DMA] * (2 * _NBUF)
        ),
    )
    def gk(table_hbm, idx_hbm, out_hbm, idx_v, *rest):
        bufs = rest[:_NBUF]
        gsem = rest[_NBUF:2 * _NBUF]
        wsem = rest[2 * _NBUF:3 * _NBUF]
        wid = lax.axis_index("s") * _NC + lax.axis_index("c")
        base = pl.multiple_of(wid * bpw, _G_CH)
        pltpu.sync_copy(idx_hbm.at[pl.ds(base, bpw)], idx_v)

        def start(j, b):
            off = pl.multiple_of(j * _G_CH, _G_CH)
            pltpu.async_copy(table_hbm.at[idx_v.at[pl.ds(off, _G_CH)]],
                             bufs[b], gsem[b])

        def gwait(b):
            pltpu.make_async_copy(table_hbm.at[pl.ds(0, _G_CH)], bufs[b],
                                  gsem[b]).wait()

        def writeback(j, b):
            off = pl.multiple_of(base + j * _G_CH, _G_CH)
            pltpu.async_copy(bufs[b], out_hbm.at[pl.ds(off, _G_CH)], wsem[b])

        def wwait(b):
            pltpu.make_async_copy(bufs[b], out_hbm.at[pl.ds(0, _G_CH)],
                                  wsem[b]).wait()

        for b in range(_NBUF):
            start(b, b)

        rounds = nch // _NBUF

        @pl.loop(0, rounds)
        def _(i):
            j0 = i * _NBUF
            for b in range(_NBUF):
                gwait(b)
                writeback(j0 + b, b)
            for b in range(_NBUF):
                @pl.when(i < rounds - 1)
                def _():
                    wwait(b)
                    start(j0 + _NBUF + b, b)

        for b in range(_NBUF):
            wwait(b)

    return gk(table, idx)


_S_CH = 128
_S_NCHTOT = _E // _S_CH
_S_NCHPAD = 1280
_S_NCH = _S_NCHPAD // _NW
_RPS = 624
_S_NBUF = 2


def _sc_segsum(e_new, idx3, zeros):
    mesh = plsc.VectorSubcoreMesh(core_axis_name="c", subcore_axis_name="s")

    @functools.partial(
        pl.kernel,
        out_type=jax.ShapeDtypeStruct((_NC, _N, _D), jnp.float32),
        mesh=mesh,
        scratch_types=(
            [pltpu.VMEM((_S_NCH, _S_CH), jnp.int32)]
            + [pltpu.VMEM((_S_CH, _D), jnp.float32)] * _S_NBUF
            + [pltpu.SemaphoreType.DMA] * (2 * _S_NBUF)
            + [pltpu.VMEM_SHARED((_N, _D), jnp.float32)]
        ),
    )
    def sk(e_hbm, idx_hbm, z_hbm, out_hbm, idx_v, *rest):
        bufs = rest[:_S_NBUF]
        lsem = rest[_S_NBUF:2 * _S_NBUF]
        ssem = rest[2 * _S_NBUF:3 * _S_NBUF]
        acc = rest[3 * _S_NBUF]
        c = lax.axis_index("c")
        s = lax.axis_index("s")
        w = c * _NS + s
        rows = pl.ds(pl.multiple_of(s * _RPS, 8), _RPS)
        tail = pl.ds(_NS * _RPS, _N - _NS * _RPS)
        pltpu.sync_copy(z_hbm.at[rows], acc.at[rows])

        @pl.when(s == _NS - 1)
        def _():
            pltpu.sync_copy(z_hbm.at[tail], acc.at[tail])

        pltpu.sync_copy(idx_hbm.at[w], idx_v)
        plsc.subcore_barrier()

        def real(j):
            return w * _S_NCH + j < _S_NCHTOT

        def startload(j, b):
            @pl.when(real(j))
            def _():
                off = pl.multiple_of((w * _S_NCH + j) * _S_CH, _S_CH)
                pltpu.async_copy(e_hbm.at[pl.ds(off, _S_CH)], bufs[b], lsem[b])

        def lwait(j, b):
            @pl.when(real(j))
            def _():
                pltpu.make_async_copy(e_hbm.at[pl.ds(0, _S_CH)], bufs[b],
                                      lsem[b]).wait()

        def scat(j, b):
            @pl.when(real(j))
            def _():
                pltpu.async_copy(bufs[b], acc.at[idx_v.at[j]], ssem[b],
                                 add=True)

        def swait(j, b):
            @pl.when(real(j))
            def _():
                pltpu.make_async_copy(e_hbm.at[pl.ds(0, _S_CH)], bufs[b],
                                      ssem[b]).wait()

        for b in range(_S_NBUF):
            startload(b, b)

        rounds = _S_NCH // _S_NBUF

        @pl.loop(0, rounds)
        def _(i):
            j0 = i * _S_NBUF
            for b in range(_S_NBUF):
                lwait(j0 + b, b)
                scat(j0 + b, b)
            for b in range(_S_NBUF):
                @pl.when(i < rounds - 1)
                def _():
                    swait(j0 + b, b)
                    startload(j0 + _S_NBUF + b, b)

        for b in range(_S_NBUF):
            swait((rounds - 1) * _S_NBUF + b, b)

        plsc.subcore_barrier()
        pltpu.sync_copy(acc.at[rows], out_hbm.at[c, rows])

        @pl.when(s == _NS - 1)
        def _():
            pltpu.sync_copy(acc.at[tail], out_hbm.at[c, tail])

    return sk(e_new, idx3, zeros)


_BLK = 1000


def _ffn(xs, offsets, rows, W1s, b1, W2, b2, W3, b3, g=None, be=None,
         res_idx=None, want_new=False):
    nx = len(xs)
    has_ln = g is not None
    dout = W3.shape[1]
    nblk = rows // _BLK

    def body(*refs):
        xr = refs[:nx]
        wr = refs[nx:2 * nx]
        b1r, w2r, b2r, w3r, b3r = refs[2 * nx:2 * nx + 5]
        k = 2 * nx + 5
        if has_ln:
            gr, ber = refs[k:k + 2]
            k += 2
        outs = refs[k:]
        acc = jnp.dot(xr[0][...], wr[0][...], preferred_element_type=jnp.float32)
        for t in range(1, nx):
            acc += jnp.dot(xr[t][...], wr[t][...], preferred_element_type=jnp.float32)
        z = jnp.maximum(acc + b1r[...], 0.0)
        z = jnp.maximum(jnp.dot(z, w2r[...], preferred_element_type=jnp.float32) + b2r[...], 0.0)
        z = jnp.dot(z, w3r[...], preferred_element_type=jnp.float32) + b3r[...]
        if has_ln:
            mu = jnp.mean(z, axis=-1, keepdims=True)
            zc = z - mu
            var = jnp.mean(zc * zc, axis=-1, keepdims=True)
            z = zc * lax.rsqrt(var + 1e-5) * gr[...] + ber[...]
        if res_idx is None:
            outs[0][...] = z
        elif want_new:
            outs[0][...] = z
            outs[1][...] = xr[res_idx][...] + z
        else:
            outs[0][...] = xr[res_idx][...] + z

    in_specs = [
        pl.BlockSpec((_BLK, x.shape[1]), functools.partial(lambda o, i: (i + o, 0), o))
        for x, o in zip(xs, offsets)
    ]
    for w in W1s:
        in_specs.append(pl.BlockSpec(w.shape, lambda i: (0, 0)))
    operands = list(xs) + list(W1s)
    for a in (b1, W2, b2, W3, b3):
        in_specs.append(pl.BlockSpec(a.shape, lambda i: (0, 0)))
        operands.append(a)
    if has_ln:
        for a in (g, be):
            in_specs.append(pl.BlockSpec(a.shape, lambda i: (0, 0)))
            operands.append(a)

    n_out = 2 if (res_idx is not None and want_new) else 1
    out_shape = [jax.ShapeDtypeStruct((rows, dout), jnp.float32)] * n_out
    out_specs = [pl.BlockSpec((_BLK, dout), lambda i: (i, 0))] * n_out
    res = pl.pallas_call(
        body,
        grid=(nblk,),
        in_specs=in_specs,
        out_specs=out_specs,
        out_shape=out_shape,
    )(*operands)
    return res if n_out > 1 else res[0]


def _prep(p):
    Ws = p['W']
    bs = [b.reshape(1, -1) for b in p['b']]
    g = p['g'].reshape(1, -1) if 'g' in p else None
    be = p['be'].reshape(1, -1) if 'be' in p else None
    return Ws, bs, g, be


def kernel(x, edge_index, edge_features, params):
    src = edge_index[0]
    dst = edge_index[1]

    Ws, bs, g, be = _prep(params['enc_node'])
    h = _ffn([x], [0], _N, [Ws[0]], bs[0], Ws[1], bs[1], Ws[2], bs[2], g, be)
    Ws, bs, g, be = _prep(params['enc_edge'])
    e = _ffn([edge_features], [0], _E, [Ws[0]], bs[0], Ws[1], bs[1], Ws[2], bs[2], g, be)

    gather_idx = jnp.concatenate([src, dst])
    dst_pad = jnp.full((_S_NCHPAD * _S_CH - _E,), _N, jnp.int32)
    dst3 = jnp.concatenate([dst, dst_pad]).reshape(_NW, _S_NCH, _S_CH)
    zeros = jnp.zeros((_N, _D), jnp.float32)

    for blk in params['proc']:
        hg = _sc_gather(h, gather_idx, 2 * _E)
        Ws, bs, g, be = _prep(blk['edge'])
        W1s = [Ws[0][:_D], Ws[0][_D:2 * _D], Ws[0][2 * _D:]]
        e_new, e = _ffn([hg, hg, e], [0, _E // _BLK, 0], _E,
                        W1s, bs[0], Ws[1], bs[1], Ws[2], bs[2], g, be,
                        res_idx=2, want_new=True)
        parts = _sc_segsum(e_new, dst3, zeros).reshape(2 * _N, _D)
        Ws, bs, g, be = _prep(blk['node'])
        W1s = [Ws[0][:_D], Ws[0][_D:], Ws[0][_D:]]
        h = _ffn([h, parts, parts], [0, 0, _N // _BLK], _N,
                 W1s, bs[0], Ws[1], bs[1], Ws[2], bs[2], g, be,
                 res_idx=0, want_new=False)

    Ws, bs, g, be = _prep(params['dec_node'])
    out_node = _ffn([h], [0], _N, [Ws[0]], bs[0], Ws[1], bs[1], Ws[2], bs[2])
    Ws, bs, g, be = _prep(params['dec_edge'])
    out_edge = _ffn([e], [0], _E, [Ws[0]], bs[0], Ws[1], bs[1], Ws[2], bs[2])
    return (out_node, out_edge)


# speedup vs baseline: 1.1142x; 1.1142x over previous
import functools

import jax
import jax.numpy as jnp
from jax import lax
from jax.experimental import pallas as pl
from jax.experimental.pallas import tpu as pltpu
from jax.experimental.pallas import tpu_sc as plsc

_N = 10000
_E = 160000
_D = 128

_NC = 2
_NS = 16
_NW = _NC * _NS


_G_CH = 80
_NBUF = 5


def _sc_gather(table, idx, m):
    bpw = m // _NW
    nch = bpw // _G_CH
    d = table.shape[1]
    dt = table.dtype
    assert bpw % _G_CH == 0 and nch % _NBUF == 0
    mesh = plsc.VectorSubcoreMesh(core_axis_name="c", subcore_axis_name="s")

    @functools.partial(
        pl.kernel,
        out_type=jax.ShapeDtypeStruct((m, d), dt),
        mesh=mesh,
        scratch_types=(
            [pltpu.VMEM((bpw,), jnp.int32)]
            + [pltpu.VMEM((_G_CH, d), dt)] * _NBUF
            + [pltpu.SemaphoreType.DMA] * (2 * _NBUF)
        ),
    )
    def gk(table_hbm, idx_hbm, out_hbm, idx_v, *rest):
        bufs = rest[:_NBUF]
        gsem = rest[_NBUF:2 * _NBUF]
        wsem = rest[2 * _NBUF:3 * _NBUF]
        wid = lax.axis_index("s") * _NC + lax.axis_index("c")
        base = pl.multiple_of(wid * bpw, _G_CH)
        pltpu.sync_copy(idx_hbm.at[pl.ds(base, bpw)], idx_v)

        def start(j, b):
            off = pl.multiple_of(j * _G_CH, _G_CH)
            pltpu.async_copy(table_hbm.at[idx_v.at[pl.ds(off, _G_CH)]],
                             bufs[b], gsem[b])

        def gwait(b):
            pltpu.make_async_copy(table_hbm.at[pl.ds(0, _G_CH)], bufs[b],
                                  gsem[b]).wait()

        def writeback(j, b):
            off = pl.multiple_of(base + j * _G_CH, _G_CH)
            pltpu.async_copy(bufs[b], out_hbm.at[pl.ds(off, _G_CH)], wsem[b])

        def wwait(b):
            pltpu.make_async_copy(bufs[b], out_hbm.at[pl.ds(0, _G_CH)],
                                  wsem[b]).wait()

        for b in range(_NBUF):
            start(b, b)

        rounds = nch // _NBUF

        @pl.loop(0, rounds)
        def _(i):
            j0 = i * _NBUF
            for b in range(_NBUF):
                gwait(b)
                writeback(j0 + b, b)
            for b in range(_NBUF):
                @pl.when(i < rounds - 1)
                def _():
                    wwait(b)
                    start(j0 + _NBUF + b, b)

        for b in range(_NBUF):
            wwait(b)

    return gk(table, idx)


_S_CH = 128
_S_NCHTOT = _E // _S_CH
_S_NCHPAD = 1280
_S_NCH = _S_NCHPAD // _NW
_RPS = 624
_S_NBUF = 2


def _sc_segsum(e_new, idx3, zeros):
    mesh = plsc.VectorSubcoreMesh(core_axis_name="c", subcore_axis_name="s")

    @functools.partial(
        pl.kernel,
        out_type=jax.ShapeDtypeStruct((_NC, _N, _D), jnp.float32),
        mesh=mesh,
        scratch_types=(
            [pltpu.VMEM((_S_NCH, _S_CH), jnp.int32)]
            + [pltpu.VMEM((_S_CH, _D), jnp.float32)] * _S_NBUF
            + [pltpu.SemaphoreType.DMA] * (2 * _S_NBUF)
            + [pltpu.VMEM_SHARED((_N, _D), jnp.float32)]
        ),
    )
    def sk(e_hbm, idx_hbm, z_hbm, out_hbm, idx_v, *rest):
        bufs = rest[:_S_NBUF]
        lsem = rest[_S_NBUF:2 * _S_NBUF]
        ssem = rest[2 * _S_NBUF:3 * _S_NBUF]
        acc = rest[3 * _S_NBUF]
        c = lax.axis_index("c")
        s = lax.axis_index("s")
        w = c * _NS + s
        rows = pl.ds(pl.multiple_of(s * _RPS, 8), _RPS)
        tail = pl.ds(_NS * _RPS, _N - _NS * _RPS)
        pltpu.sync_copy(z_hbm.at[rows], acc.at[rows])

        @pl.when(s == _NS - 1)
        def _():
            pltpu.sync_copy(z_hbm.at[tail], acc.at[tail])

        pltpu.sync_copy(idx_hbm.at[w], idx_v)
        plsc.subcore_barrier()

        def real(j):
            return w * _S_NCH + j < _S_NCHTOT

        def startload(j, b):
            @pl.when(real(j))
            def _():
                off = pl.multiple_of((w * _S_NCH + j) * _S_CH, _S_CH)
                pltpu.async_copy(e_hbm.at[pl.ds(off, _S_CH)], bufs[b], lsem[b])

        def lwait(j, b):
            @pl.when(real(j))
            def _():
                pltpu.make_async_copy(e_hbm.at[pl.ds(0, _S_CH)], bufs[b],
                                      lsem[b]).wait()

        def scat(j, b):
            @pl.when(real(j))
            def _():
                pltpu.async_copy(bufs[b], acc.at[idx_v.at[j]], ssem[b],
                                 add=True)

        def swait(j, b):
            @pl.when(real(j))
            def _():
                pltpu.make_async_copy(e_hbm.at[pl.ds(0, _S_CH)], bufs[b],
                                      ssem[b]).wait()

        for b in range(_S_NBUF):
            startload(b, b)

        rounds = _S_NCH // _S_NBUF

        @pl.loop(0, rounds)
        def _(i):
            j0 = i * _S_NBUF
            for b in range(_S_NBUF):
                lwait(j0 + b, b)
                scat(j0 + b, b)
            for b in range(_S_NBUF):
                @pl.when(i < rounds - 1)
                def _():
                    swait(j0 + b, b)
                    startload(j0 + _S_NBUF + b, b)

        for b in range(_S_NBUF):
            swait((rounds - 1) * _S_NBUF + b, b)

        plsc.subcore_barrier()
        pltpu.sync_copy(acc.at[rows], out_hbm.at[c, rows])

        @pl.when(s == _NS - 1)
        def _():
            pltpu.sync_copy(acc.at[tail], out_hbm.at[c, tail])

    return sk(e_new, idx3, zeros)


_BLK = 1000


def _bdot(a, w):
    return jnp.dot(a.astype(jnp.bfloat16), w.astype(jnp.bfloat16),
                   preferred_element_type=jnp.float32)


def _fdot(a, w):
    return jnp.dot(a, w, preferred_element_type=jnp.float32)


def _ffn(xs, offsets, rows, W1s, b1, W2, b2, W3, b3, g=None, be=None,
         res_idx=None, want_new=False):
    nx = len(xs)
    has_ln = g is not None
    dout = W3.shape[1]
    nblk = rows // _BLK

    def body(*refs):
        xr = refs[:nx]
        wr = refs[nx:2 * nx]
        b1r, w2r, b2r, w3r, b3r = refs[2 * nx:2 * nx + 5]
        k = 2 * nx + 5
        if has_ln:
            gr, ber = refs[k:k + 2]
            k += 2
        outs = refs[k:]
        acc = _fdot(xr[0][...], wr[0][...])
        for t in range(1, nx):
            acc += _fdot(xr[t][...], wr[t][...])
        z = jnp.maximum(acc + b1r[...], 0.0)
        z = jnp.maximum(_fdot(z, w2r[...]) + b2r[...], 0.0)
        z = _fdot(z, w3r[...]) + b3r[...]
        if has_ln:
            mu = jnp.mean(z, axis=-1, keepdims=True)
            zc = z - mu
            var = jnp.mean(zc * zc, axis=-1, keepdims=True)
            z = zc * lax.rsqrt(var + 1e-5) * gr[...] + ber[...]
        if res_idx is None:
            outs[0][...] = z
        elif want_new:
            outs[0][...] = z
            outs[1][...] = xr[res_idx][...] + z
        else:
            outs[0][...] = xr[res_idx][...] + z

    in_specs = [
        pl.BlockSpec((_BLK, x.shape[1]), functools.partial(lambda o, i: (i + o, 0), o))
        for x, o in zip(xs, offsets)
    ]
    for w in W1s:
        in_specs.append(pl.BlockSpec(w.shape, lambda i: (0, 0)))
    operands = list(xs) + list(W1s)
    for a in (b1, W2, b2, W3, b3):
        in_specs.append(pl.BlockSpec(a.shape, lambda i: (0, 0)))
        operands.append(a)
    if has_ln:
        for a in (g, be):
            in_specs.append(pl.BlockSpec(a.shape, lambda i: (0, 0)))
            operands.append(a)

    n_out = 2 if (res_idx is not None and want_new) else 1
    out_shape = [jax.ShapeDtypeStruct((rows, dout), jnp.float32)] * n_out
    out_specs = [pl.BlockSpec((_BLK, dout), lambda i: (i, 0))] * n_out
    res = pl.pallas_call(
        body,
        grid=(nblk,),
        in_specs=in_specs,
        out_specs=out_specs,
        out_shape=out_shape,
    )(*operands)
    return res if n_out > 1 else res[0]


_EBLK = 2000


def _edge_mlp(hgp, e, W1s, b1, W2, b2, W3, b3, g, be):
    nblk = _E // _EBLK

    def body(hr, er, w1a, w1b, w1c, b1r, w2r, b2r, w3r, b3r, gr, ber,
             out_new, out_next):
        acc = _bdot(hr[0], w1a[...]) + _bdot(hr[1], w1b[...])
        acc += _fdot(er[...], w1c[...])
        z = jnp.maximum(acc + b1r[...], 0.0)
        z = jnp.maximum(_fdot(z, w2r[...]) + b2r[...], 0.0)
        z = _fdot(z, w3r[...]) + b3r[...]
        mu = jnp.mean(z, axis=-1, keepdims=True)
        zc = z - mu
        var = jnp.mean(zc * zc, axis=-1, keepdims=True)
        z = zc * lax.rsqrt(var + 1e-5) * gr[...] + ber[...]
        out_new[...] = z
        out_next[...] = er[...] + z

    in_specs = [pl.BlockSpec((2, _EBLK, _D), lambda i: (0, i, 0)),
                pl.BlockSpec((_EBLK, _D), lambda i: (i, 0))]
    operands = [hgp, e]
    for a in (W1s[0], W1s[1], W1s[2], b1, W2, b2, W3, b3, g, be):
        in_specs.append(pl.BlockSpec(a.shape, lambda i: (0, 0)))
        operands.append(a)
    out_shape = [jax.ShapeDtypeStruct((_E, _D), jnp.float32)] * 2
    out_specs = [pl.BlockSpec((_EBLK, _D), lambda i: (i, 0))] * 2
    return pl.pallas_call(
        body,
        grid=(nblk,),
        in_specs=in_specs,
        out_specs=out_specs,
        out_shape=out_shape,
    )(*operands)


def _prep(p):
    Ws = p['W']
    bs = [b.reshape(1, -1) for b in p['b']]
    g = p['g'].reshape(1, -1) if 'g' in p else None
    be = p['be'].reshape(1, -1) if 'be' in p else None
    return Ws, bs, g, be


def kernel(x, edge_index, edge_features, params):
    src = edge_index[0]
    dst = edge_index[1]

    Ws, bs, g, be = _prep(params['enc_node'])
    h = _ffn([x], [0], _N, [Ws[0]], bs[0], Ws[1], bs[1], Ws[2], bs[2], g, be)
    Ws, bs, g, be = _prep(params['enc_edge'])
    e = _ffn([edge_features], [0], _E, [Ws[0]], bs[0], Ws[1], bs[1], Ws[2], bs[2], g, be)

    gather_idx = jnp.concatenate([src, dst])
    dst_pad = jnp.full((_S_NCHPAD * _S_CH - _E,), _N, jnp.int32)
    dst3 = jnp.concatenate([dst, dst_pad]).reshape(_NW, _S_NCH, _S_CH)
    zeros = jnp.zeros((_N, _D), jnp.float32)

    for blk in params['proc']:
        hg = _sc_gather(h, gather_idx, 2 * _E)
        hgp = hg.reshape(2, _E, _D)
        Ws, bs, g, be = _prep(blk['edge'])
        W1s = [Ws[0][:_D], Ws[0][_D:2 * _D], Ws[0][2 * _D:]]
        e_new, e = _edge_mlp(hgp, e, W1s, bs[0], Ws[1], bs[1], Ws[2], bs[2],
                             g, be)
        parts = _sc_segsum(e_new, dst3, zeros).reshape(2 * _N, _D)
        Ws, bs, g, be = _prep(blk['node'])
        W1s = [Ws[0][:_D], Ws[0][_D:], Ws[0][_D:]]
        h = _ffn([h, parts, parts], [0, 0, _N // _BLK], _N,
                 W1s, bs[0], Ws[1], bs[1], Ws[2], bs[2], g, be,
                 res_idx=0, want_new=False)

    Ws, bs, g, be = _prep(params['dec_node'])
    out_node = _ffn([h], [0], _N, [Ws[0]], bs[0], Ws[1], bs[1], Ws[2], bs[2])
    Ws, bs, g, be = _prep(params['dec_edge'])
    out_edge = _ffn([e], [0], _E, [Ws[0]], bs[0], Ws[1], bs[1], Ws[2], bs[2])
    return (out_node, out_edge)


# speedup vs baseline: 1.1370x; 1.0205x over previous
import functools

import jax
import jax.numpy as jnp
from jax import lax
from jax.experimental import pallas as pl
from jax.experimental.pallas import tpu as pltpu
from jax.experimental.pallas import tpu_sc as plsc

_N = 10000
_E = 160000
_D = 128

_NC = 2
_NS = 16
_NW = _NC * _NS


_G_CH = 80
_NBUF = 5


def _sc_gather(table, idx, m):
    bpw = m // _NW
    nch = bpw // _G_CH
    d = table.shape[1]
    dt = table.dtype
    assert bpw % _G_CH == 0 and nch % _NBUF == 0
    mesh = plsc.VectorSubcoreMesh(core_axis_name="c", subcore_axis_name="s")

    @functools.partial(
        pl.kernel,
        out_type=jax.ShapeDtypeStruct((m, d), dt),
        mesh=mesh,
        scratch_types=(
            [pltpu.VMEM((bpw,), jnp.int32)]
            + [pltpu.VMEM((_G_CH, d), dt)] * _NBUF
            + [pltpu.SemaphoreType.DMA] * (2 * _NBUF)
        ),
    )
    def gk(table_hbm, idx_hbm, out_hbm, idx_v, *rest):
        bufs = rest[:_NBUF]
        gsem = rest[_NBUF:2 * _NBUF]
        wsem = rest[2 * _NBUF:3 * _NBUF]
        wid = lax.axis_index("s") * _NC + lax.axis_index("c")
        base = pl.multiple_of(wid * bpw, _G_CH)
        pltpu.sync_copy(idx_hbm.at[pl.ds(base, bpw)], idx_v)

        def start(j, b):
            off = pl.multiple_of(j * _G_CH, _G_CH)
            pltpu.async_copy(table_hbm.at[idx_v.at[pl.ds(off, _G_CH)]],
                             bufs[b], gsem[b])

        def gwait(b):
            pltpu.make_async_copy(table_hbm.at[pl.ds(0, _G_CH)], bufs[b],
                                  gsem[b]).wait()

        def writeback(j, b):
            off = pl.multiple_of(base + j * _G_CH, _G_CH)
            pltpu.async_copy(bufs[b], out_hbm.at[pl.ds(off, _G_CH)], wsem[b])

        def wwait(b):
            pltpu.make_async_copy(bufs[b], out_hbm.at[pl.ds(0, _G_CH)],
                                  wsem[b]).wait()

        for b in range(_NBUF):
            start(b, b)

        rounds = nch // _NBUF

        @pl.loop(0, rounds)
        def _(i):
            j0 = i * _NBUF
            for b in range(_NBUF):
                gwait(b)
                writeback(j0 + b, b)
            for b in range(_NBUF):
                @pl.when(i < rounds - 1)
                def _():
                    wwait(b)
                    start(j0 + _NBUF + b, b)

        for b in range(_NBUF):
            wwait(b)

    return gk(table, idx)


_S_CH = 128
_S_NCHTOT = _E // _S_CH
_S_NCHPAD = 1280
_S_NCH = _S_NCHPAD // _NW
_RPS = 624
_S_NBUF = 2


def _sc_segsum(e_new, idx3, zeros):
    mesh = plsc.VectorSubcoreMesh(core_axis_name="c", subcore_axis_name="s")

    @functools.partial(
        pl.kernel,
        out_type=jax.ShapeDtypeStruct((_NC, _N, _D), jnp.float32),
        mesh=mesh,
        scratch_types=(
            [pltpu.VMEM((_S_NCH, _S_CH), jnp.int32)]
            + [pltpu.VMEM((_S_CH, _D), jnp.float32)] * _S_NBUF
            + [pltpu.SemaphoreType.DMA] * (2 * _S_NBUF)
            + [pltpu.VMEM_SHARED((_N, _D), jnp.float32)]
        ),
    )
    def sk(e_hbm, idx_hbm, z_hbm, out_hbm, idx_v, *rest):
        bufs = rest[:_S_NBUF]
        lsem = rest[_S_NBUF:2 * _S_NBUF]
        ssem = rest[2 * _S_NBUF:3 * _S_NBUF]
        acc = rest[3 * _S_NBUF]
        c = lax.axis_index("c")
        s = lax.axis_index("s")
        w = c * _NS + s
        rows = pl.ds(pl.multiple_of(s * _RPS, 8), _RPS)
        tail = pl.ds(_NS * _RPS, _N - _NS * _RPS)
        pltpu.sync_copy(z_hbm.at[rows], acc.at[rows])

        @pl.when(s == _NS - 1)
        def _():
            pltpu.sync_copy(z_hbm.at[tail], acc.at[tail])

        pltpu.sync_copy(idx_hbm.at[w], idx_v)
        plsc.subcore_barrier()

        def real(j):
            return w * _S_NCH + j < _S_NCHTOT

        def startload(j, b):
            @pl.when(real(j))
            def _():
                off = pl.multiple_of((w * _S_NCH + j) * _S_CH, _S_CH)
                pltpu.async_copy(e_hbm.at[pl.ds(off, _S_CH)], bufs[b], lsem[b])

        def lwait(j, b):
            @pl.when(real(j))
            def _():
                pltpu.make_async_copy(e_hbm.at[pl.ds(0, _S_CH)], bufs[b],
                                      lsem[b]).wait()

        def scat(j, b):
            @pl.when(real(j))
            def _():
                pltpu.async_copy(bufs[b], acc.at[idx_v.at[j]], ssem[b],
                                 add=True)

        def swait(j, b):
            @pl.when(real(j))
            def _():
                pltpu.make_async_copy(e_hbm.at[pl.ds(0, _S_CH)], bufs[b],
                                      ssem[b]).wait()

        for b in range(_S_NBUF):
            startload(b, b)

        rounds = _S_NCH // _S_NBUF

        @pl.loop(0, rounds)
        def _(i):
            j0 = i * _S_NBUF
            for b in range(_S_NBUF):
                lwait(j0 + b, b)
                scat(j0 + b, b)
            for b in range(_S_NBUF):
                @pl.when(i < rounds - 1)
                def _():
                    swait(j0 + b, b)
                    startload(j0 + _S_NBUF + b, b)

        for b in range(_S_NBUF):
            swait((rounds - 1) * _S_NBUF + b, b)

        plsc.subcore_barrier()
        pltpu.sync_copy(acc.at[rows], out_hbm.at[c, rows])

        @pl.when(s == _NS - 1)
        def _():
            pltpu.sync_copy(acc.at[tail], out_hbm.at[c, tail])

    return sk(e_new, idx3, zeros)


_BLK = 1000


def _bdot(a, w):
    return jnp.dot(a.astype(jnp.bfloat16), w.astype(jnp.bfloat16),
                   preferred_element_type=jnp.float32)


def _fdot(a, w):
    return jnp.dot(a, w, preferred_element_type=jnp.float32)


def _ffn(xs, offsets, rows, W1s, b1, W2, b2, W3, b3, g=None, be=None,
         res_idx=None, want_new=False):
    nx = len(xs)
    has_ln = g is not None
    dout = W3.shape[1]
    nblk = rows // _BLK

    def body(*refs):
        xr = refs[:nx]
        wr = refs[nx:2 * nx]
        b1r, w2r, b2r, w3r, b3r = refs[2 * nx:2 * nx + 5]
        k = 2 * nx + 5
        if has_ln:
            gr, ber = refs[k:k + 2]
            k += 2
        outs = refs[k:]
        acc = _fdot(xr[0][...], wr[0][...])
        for t in range(1, nx):
            acc += _fdot(xr[t][...], wr[t][...])
        z = jnp.maximum(acc + b1r[...], 0.0)
        z = jnp.maximum(_fdot(z, w2r[...]) + b2r[...], 0.0)
        z = _fdot(z, w3r[...]) + b3r[...]
        if has_ln:
            mu = jnp.mean(z, axis=-1, keepdims=True)
            zc = z - mu
            var = jnp.mean(zc * zc, axis=-1, keepdims=True)
            z = zc * lax.rsqrt(var + 1e-5) * gr[...] + ber[...]
        if res_idx is None:
            outs[0][...] = z
        elif want_new:
            outs[0][...] = z
            outs[1][...] = xr[res_idx][...] + z
        else:
            outs[0][...] = xr[res_idx][...] + z

    in_specs = [
        pl.BlockSpec((_BLK, x.shape[1]), functools.partial(lambda o, i: (i + o, 0), o))
        for x, o in zip(xs, offsets)
    ]
    for w in W1s:
        in_specs.append(pl.BlockSpec(w.shape, lambda i: (0, 0)))
    operands = list(xs) + list(W1s)
    for a in (b1, W2, b2, W3, b3):
        in_specs.append(pl.BlockSpec(a.shape, lambda i: (0, 0)))
        operands.append(a)
    if has_ln:
        for a in (g, be):
            in_specs.append(pl.BlockSpec(a.shape, lambda i: (0, 0)))
            operands.append(a)

    n_out = 2 if (res_idx is not None and want_new) else 1
    out_shape = [jax.ShapeDtypeStruct((rows, dout), jnp.float32)] * n_out
    out_specs = [pl.BlockSpec((_BLK, dout), lambda i: (i, 0))] * n_out
    res = pl.pallas_call(
        body,
        grid=(nblk,),
        in_specs=in_specs,
        out_specs=out_specs,
        out_shape=out_shape,
    )(*operands)
    return res if n_out > 1 else res[0]


_EBLK = 2000


def _edge_mlp(hgp, e, W1s, b1, W2, b2, W3, b3, g, be):
    nblk = _E // _EBLK

    def body(hr, er, w1a, w1b, w1c, b1r, w2r, b2r, w3r, b3r, gr, ber,
             out_new, out_next):
        acc = _fdot(hr[0], w1a[...]) + _fdot(hr[1], w1b[...])
        acc += _fdot(er[...], w1c[...])
        z = jnp.maximum(acc + b1r[...], 0.0)
        z = jnp.maximum(_fdot(z, w2r[...]) + b2r[...], 0.0)
        z = _fdot(z, w3r[...]) + b3r[...]
        mu = jnp.mean(z, axis=-1, keepdims=True)
        zc = z - mu
        var = jnp.mean(zc * zc, axis=-1, keepdims=True)
        z = zc * lax.rsqrt(var + 1e-5) * gr[...] + ber[...]
        out_new[...] = z
        out_next[...] = er[...] + z

    in_specs = [pl.BlockSpec((2, _EBLK, _D), lambda i: (0, i, 0)),
                pl.BlockSpec((_EBLK, _D), lambda i: (i, 0))]
    operands = [hgp, e]
    for a in (W1s[0], W1s[1], W1s[2], b1, W2, b2, W3, b3, g, be):
        in_specs.append(pl.BlockSpec(a.shape, lambda i: (0, 0)))
        operands.append(a)
    out_shape = [jax.ShapeDtypeStruct((_E, _D), jnp.float32)] * 2
    out_specs = [pl.BlockSpec((_EBLK, _D), lambda i: (i, 0))] * 2
    return pl.pallas_call(
        body,
        grid=(nblk,),
        in_specs=in_specs,
        out_specs=out_specs,
        out_shape=out_shape,
    )(*operands)


def _prep(p):
    Ws = p['W']
    bs = [b.reshape(1, -1) for b in p['b']]
    g = p['g'].reshape(1, -1) if 'g' in p else None
    be = p['be'].reshape(1, -1) if 'be' in p else None
    return Ws, bs, g, be


def kernel(x, edge_index, edge_features, params):
    src = edge_index[0]
    dst = edge_index[1]

    Ws, bs, g, be = _prep(params['enc_node'])
    h = _ffn([x], [0], _N, [Ws[0]], bs[0], Ws[1], bs[1], Ws[2], bs[2], g, be)
    Ws, bs, g, be = _prep(params['enc_edge'])
    e = _ffn([edge_features], [0], _E, [Ws[0]], bs[0], Ws[1], bs[1], Ws[2], bs[2], g, be)

    gather_idx = jnp.concatenate([src, dst])
    dst_pad = jnp.full((_S_NCHPAD * _S_CH - _E,), _N, jnp.int32)
    dst3 = jnp.concatenate([dst, dst_pad]).reshape(_NW, _S_NCH, _S_CH)
    zeros = jnp.zeros((_N, _D), jnp.float32)

    for blk in params['proc']:
        hg = _sc_gather(h, gather_idx, 2 * _E)
        hgp = hg.reshape(2, _E, _D)
        Ws, bs, g, be = _prep(blk['edge'])
        W1s = [Ws[0][:_D], Ws[0][_D:2 * _D], Ws[0][2 * _D:]]
        e_new, e = _edge_mlp(hgp, e, W1s, bs[0], Ws[1], bs[1], Ws[2], bs[2],
                             g, be)
        parts = _sc_segsum(e_new, dst3, zeros).reshape(2 * _N, _D)
        Ws, bs, g, be = _prep(blk['node'])
        W1s = [Ws[0][:_D], Ws[0][_D:], Ws[0][_D:]]
        h = _ffn([h, parts, parts], [0, 0, _N // _BLK], _N,
                 W1s, bs[0], Ws[1], bs[1], Ws[2], bs[2], g, be,
                 res_idx=0, want_new=False)

    Ws, bs, g, be = _prep(params['dec_node'])
    out_node = _ffn([h], [0], _N, [Ws[0]], bs[0], Ws[1], bs[1], Ws[2], bs[2])
    Ws, bs, g, be = _prep(params['dec_edge'])
    out_edge = _ffn([e], [0], _E, [Ws[0]], bs[0], Ws[1], bs[1], Ws[2], bs[2])
    return (out_node, out_edge)
